# baseline (device time: 30731 ns/iter reference)
import jax
import jax.numpy as jnp
from jax import lax
from jax.experimental import pallas as pl
from jax.experimental.pallas import tpu as pltpu

N_DEV = 32


def kernel(Q, K, V):
    b, q_len, h, d = Q.shape
    kv_len = K.shape[1]
    hd = h * d
    scale = d ** -0.5

    pack = 640

    Q2 = Q.reshape(b, hd)
    K2 = K.reshape(b, kv_len, hd)
    V2 = V.reshape(b, kv_len, hd)

    def body(q_ref, k_ref, v_ref, o_ref, send_ref, allrecv_ref, send_sems, recv_sems):
        my = lax.axis_index("i")

        f32 = jnp.float32
        E = (lax.broadcasted_iota(jnp.int32, (hd, h), 0) // d
             == lax.broadcasted_iota(jnp.int32, (hd, h), 1)).astype(f32)
        ET = (lax.broadcasted_iota(jnp.int32, (h, hd), 0)
              == lax.broadcasted_iota(jnp.int32, (h, hd), 1) // d).astype(f32)

        dot = lambda a, c: lax.dot_general(
            a, c, (((1,), (0,)), ((), ())), preferred_element_type=f32)

        qv = q_ref[...]
        W_all = qv[:, :, None] * E[None, :, :]

        o_rows, m_rows, l_rows = [], [], []
        for bi in range(b):
            Sb = dot(k_ref[bi], W_all[bi]) * scale
            mb = jnp.max(Sb, axis=0, keepdims=True)
            pb = jnp.exp(Sb - mb)
            lb = jnp.sum(pb, axis=0, keepdims=True)
            P2 = dot(pb, ET)
            Ob = jnp.sum(P2 * v_ref[bi], axis=0, keepdims=True)
            o_rows.append(Ob)
            m_rows.append(mb)
            l_rows.append(lb)

        o = jnp.concatenate(o_rows, axis=0)
        m = jnp.concatenate(m_rows, axis=0)
        l = jnp.concatenate(l_rows, axis=0)

        send_ref[:, 0:hd] = o.astype(jnp.bfloat16)
        send_ref[:, hd:hd + h] = m.astype(jnp.bfloat16)
        send_ref[:, hd + h:hd + 2 * h] = l.astype(jnp.bfloat16)

        barrier_sem = pltpu.get_barrier_semaphore()
        for off in range(1, N_DEV):
            pl.semaphore_signal(
                barrier_sem,
                inc=1,
                device_id=((my + off) % N_DEV,),
                device_id_type=pl.DeviceIdType.MESH,
            )
        pl.semaphore_wait(barrier_sem, N_DEV - 1)

        sends = []
        for off in range(1, N_DEV):
            rdma = pltpu.make_async_remote_copy(
                src_ref=send_ref,
                dst_ref=allrecv_ref.at[my],
                send_sem=send_sems.at[off],
                recv_sem=recv_sems.at[my],
                device_id=((my + off) % N_DEV,),
                device_id_type=pl.DeviceIdType.MESH,
            )
            rdma.start()
            sends.append(rdma)

        allrecv_ref[my] = send_ref[...]

        for off in range(1, N_DEV):
            src = (my + off) % N_DEV
            recv = pltpu.make_async_remote_copy(
                src_ref=send_ref,
                dst_ref=allrecv_ref.at[src],
                send_sem=send_sems.at[off],
                recv_sem=recv_sems.at[src],
                device_id=(src,),
                device_id_type=pl.DeviceIdType.MESH,
            )
            recv.wait_recv()

        data = allrecv_ref[...]
        om = data[:, :, 0:hd].astype(f32)
        mm = data[:, :, hd:hd + h].astype(f32)
        lm = data[:, :, hd + h:hd + 2 * h].astype(f32)

        mg = jnp.max(mm, axis=0)
        a = jnp.exp(mm - mg[None, :, :])
        lg = jnp.sum(lm * a, axis=0)
        A = dot(a.reshape(N_DEV * b, h), ET)
        og = jnp.sum(om * A.reshape(N_DEV, b, hd), axis=0)
        o_ref[...] = og / dot(lg, ET)

        for rdma in sends:
            rdma.wait_send()

    out2d = pl.pallas_call(
        body,
        out_shape=jax.ShapeDtypeStruct((b, hd), jnp.float32),
        in_specs=[
            pl.BlockSpec(memory_space=pltpu.VMEM),
            pl.BlockSpec(memory_space=pltpu.VMEM),
            pl.BlockSpec(memory_space=pltpu.VMEM),
        ],
        out_specs=pl.BlockSpec(memory_space=pltpu.VMEM),
        scratch_shapes=[
            pltpu.VMEM((b, pack), jnp.bfloat16),
            pltpu.VMEM((N_DEV, b, pack), jnp.bfloat16),
            pltpu.SemaphoreType.DMA((N_DEV,)),
            pltpu.SemaphoreType.DMA((N_DEV,)),
        ],
        compiler_params=pltpu.CompilerParams(collective_id=0),
    )(Q2, K2, V2)
    return out2d.reshape(b, q_len, h, d)
